# baseline (speedup 1.0000x reference)
import functools

import jax
import jax.numpy as jnp
from jax import lax
from jax.experimental import pallas as pl
from jax.experimental.pallas import tpu as pltpu
from jax.experimental.pallas import tpu_sc as plsc

B = 16384
D = 64
OUT_D = 4 * D
N_NODES = 1000000
NUM_CORES = 2
NUM_SUBCORES = 16
NW = NUM_CORES * NUM_SUBCORES

WCOLS = 31232
WIN = 512
NWIN = WCOLS // WIN
SUB = 4096
NSUB = 8
CAP1 = 704
CAP2 = 160
DUMP = B
TAILA = NW * WCOLS
TAILB = TAILA + WIN


def _iota16():
    return lax.broadcasted_iota(jnp.int32, (16,), 0)


@functools.partial(
    pl.kernel,
    mesh=plsc.VectorSubcoreMesh(core_axis_name="c", subcore_axis_name="s"),
    out_type=jax.ShapeDtypeStruct((B + 16, 128), jnp.float32),
    scratch_types=[
        pltpu.VMEM((B,), jnp.int32),
        pltpu.VMEM((CAP1 + 16,), jnp.int32),
        pltpu.VMEM((CAP1 + 16,), jnp.int32),
        pltpu.VMEM((CAP2 + 16,), jnp.int32),
        pltpu.VMEM((CAP2 + 16,), jnp.int32),
        pltpu.VMEM((CAP2 + 16,), jnp.int32),
        pltpu.VMEM((CAP2 + 16,), jnp.int32),
        pltpu.VMEM((D, WIN), jnp.float32),
        pltpu.VMEM((D, WIN), jnp.float32),
        pltpu.VMEM((D, D), jnp.float32),
        pltpu.VMEM((16, 128), jnp.float32),
        pltpu.VMEM((16, 128), jnp.float32),
        pltpu.VMEM((16, 128), jnp.float32),
        pltpu.VMEM((16,), jnp.int32),
        pltpu.VMEM((16,), jnp.int32),
        pltpu.VMEM((16,), jnp.int32),
        pltpu.SemaphoreType.DMA,
        pltpu.SemaphoreType.DMA,
        pltpu.SemaphoreType.DMA,
        pltpu.SemaphoreType.DMA,
        pltpu.SemaphoreType.DMA,
    ],
    compiler_params=pltpu.CompilerParams(needs_layout_passes=False),
)
def _sc_stream_gather(idx_hbm, tableT, tailT, pos, idx_v, h1i_v, h1j_v,
                      h2i_v, h2j_v, h3i_v, h3j_v, slabA, slabB, tail_v,
                      rbA, rbB, rbC, jlA, jlB, jlC, semA, semB, semS, semS2, semC):
    wid = lax.axis_index("s") * NUM_CORES + lax.axis_index("c")
    wbase = wid * WCOLS
    lo = wbase
    hi = jnp.where(wid == NW - 1, N_NODES, wbase + WCOLS)

    slabs = [slabA, slabB]
    sems = [semA, semB]
    slab_h = [None, None]
    for k in range(2):
        col = pl.multiple_of(wbase + k * WIN, WIN)
        slab_h[k] = pltpu.async_copy(
            tableT.at[:, pl.ds(col, WIN)], slabs[k], sems[k])

    pltpu.sync_copy(idx_hbm, idx_v)

    def l1_body(i, cnt):
        v = idx_v[pl.ds(i * 16, 16)]
        jv = _iota16() + i * 16
        m = (v >= lo) & (v < hi)
        npop = plsc.all_reduce_population_count(m)[0]

        @pl.when(npop > 0)
        def _():
            plsc.store_compressed(h1i_v.at[pl.ds(cnt, 16)], v, mask=m)
            plsc.store_compressed(h1j_v.at[pl.ds(cnt, 16)], jv, mask=m)

        return cnt + npop

    cnt1 = lax.fori_loop(0, B // 16, l1_body, 0)

    rbufs = [rbA, rbB]
    jls = [jlA, jlB]
    scat_h = [None, None]

    def compress_window(gcol_lo, cnt2):
        def l3_body(i, cnt):
            v = h2i_v[pl.ds(i * 16, 16)]
            jv = h2j_v[pl.ds(i * 16, 16)]
            m = (((_iota16() + i * 16) < cnt2)
                 & (v >= gcol_lo) & (v < gcol_lo + WIN))
            npop = plsc.all_reduce_population_count(m)[0]

            @pl.when(npop > 0)
            def _():
                plsc.store_compressed(h3i_v.at[pl.ds(cnt, 16)], v, mask=m)
                plsc.store_compressed(h3j_v.at[pl.ds(cnt, 16)], jv, mask=m)

            return cnt + npop

        return lax.fori_loop(0, (cnt2 + 15) >> 4, l3_body, 0)

    def build_row_group(slab, fetch_lo, off, cnt3, rb):
        hv = h3i_v[pl.ds(off, 16)]
        jv = h3j_v[pl.ds(off, 16)]
        validm = (_iota16() + off) < cnt3
        lvec = jnp.where(validm, hv - fetch_lo, 0)

        def d_body(d, _):
            dv = jnp.full((16,), d, jnp.int32)
            vals = plsc.load_gather(slab, [dv, lvec], mask=validm)
            plsc.store_scatter(rb, [_iota16(), dv], vals)
            return 0

        lax.fori_loop(0, D, d_body, 0)
        return jnp.where(validm, jv, DUMP)

    def extract_async(slab, fetch_lo, cnt3, rb, jl, sem):
        pltpu.make_async_copy(rb, pos.at[jl], sem).wait()
        jdst = build_row_group(slab, fetch_lo, 0, cnt3, rb)
        jl[...] = jdst
        pltpu.async_copy(rb, pos.at[jl], sem)

    def extract_sync(slab, fetch_lo, off, cnt3):
        jdst = build_row_group(slab, fetch_lo, off, cnt3, rbC)
        jlC[...] = jdst
        pltpu.async_copy(rbC, pos.at[jlC], semC).wait()

    def rare_groups(slab, fetch_lo, cnt3, first=1):
        def body(i, _):
            extract_sync(slab, fetch_lo, i * 16, cnt3)
            return 0

        lax.fori_loop(first, (cnt3 + 15) >> 4, body, 0)

    def process_window(slab, gcol, fetch_lo, cnt2, rb, jl, sem):
        cnt3 = compress_window(gcol, cnt2)
        extract_async(slab, fetch_lo, cnt3, rb, jl, sem)
        rare_groups(slab, fetch_lo, cnt3)

    jlA[...] = jnp.full((16,), DUMP, jnp.int32)
    jlB[...] = jnp.full((16,), DUMP, jnp.int32)
    pltpu.async_copy(rbA, pos.at[jlA], semS)
    pltpu.async_copy(rbB, pos.at[jlB], semS2)

    def sub_count(s):
        slo = wbase + s * SUB
        shi = jnp.minimum(slo + SUB, hi)

        def l2_body(i, cnt):
            v = h1i_v[pl.ds(i * 16, 16)]
            jv = h1j_v[pl.ds(i * 16, 16)]
            m = (((_iota16() + i * 16) < cnt1) & (v >= slo) & (v < shi))
            npop = plsc.all_reduce_population_count(m)[0]

            @pl.when(npop > 0)
            def _():
                plsc.store_compressed(h2i_v.at[pl.ds(cnt, 16)], v, mask=m)
                plsc.store_compressed(h2j_v.at[pl.ds(cnt, 16)], jv, mask=m)

            return cnt + npop

        return lax.fori_loop(0, (cnt1 + 15) >> 4, l2_body, 0)

    def pair_body(t, _):
        cnt2 = sub_count(t >> 2)
        g0 = 2 * t
        colA = pl.multiple_of(wbase + g0 * WIN, WIN)
        pltpu.make_async_copy(
            tableT.at[:, pl.ds(colA, WIN)], slabA, semA).wait()
        process_window(slabA, colA, colA, cnt2, rbA, jlA, semS)

        @pl.when(g0 + 2 < NWIN)
        def _():
            col = pl.multiple_of(wbase + (g0 + 2) * WIN, WIN)
            pltpu.async_copy(tableT.at[:, pl.ds(col, WIN)], slabA, semA)

        colB = pl.multiple_of(wbase + (g0 + 1) * WIN, WIN)
        pltpu.make_async_copy(
            tableT.at[:, pl.ds(colB, WIN)], slabB, semB).wait()
        process_window(slabB, colB, colB, cnt2, rbB, jlB, semS2)

        @pl.when(g0 + 3 < NWIN)
        def _():
            col = pl.multiple_of(wbase + (g0 + 3) * WIN, WIN)
            pltpu.async_copy(tableT.at[:, pl.ds(col, WIN)], slabB, semB)

        return 0

    lax.fori_loop(0, NWIN // 2, pair_body, 0)
    cnt2 = sub_count(NSUB - 1)

    colA = pl.multiple_of(wbase + (NWIN - 1) * WIN, WIN)
    pltpu.make_async_copy(
        tableT.at[:, pl.ds(colA, WIN)], slabA, semA).wait()
    cnt3 = compress_window(colA, cnt2)
    rare_groups(slabA, colA, cnt3, first=0)

    @pl.when(wid == NW - 1)
    def _():
        pltpu.async_copy(
            tableT.at[:, pl.ds(TAILA, WIN)], slabA, semA).wait()
        cnt3t = compress_window(TAILA, cnt2)
        rare_groups(slabA, TAILA, cnt3t, first=0)

    @pl.when(wid == NW - 1)
    def _():
        pltpu.async_copy(tailT, tail_v, semB).wait()
        cnt3t = compress_window(TAILB, cnt2)
        rare_groups(tail_v, TAILB, cnt3t, first=0)

    pltpu.make_async_copy(rbA, pos.at[jlA], semS).wait()
    pltpu.make_async_copy(rbB, pos.at[jlB], semS2).wait()


def _concat_body(z_src_ref, z_dst_ref, pos_ref, t_ref, out_ref):
    out_ref[...] = jnp.concatenate(
        [z_src_ref[...], z_dst_ref[...], pos_ref[...][:, :D], t_ref[...]],
        axis=-1)


_R = 2048
_concat = pl.pallas_call(
    _concat_body,
    grid=(B // _R,),
    in_specs=[pl.BlockSpec((_R, D), lambda i: (i, 0))] * 2
    + [pl.BlockSpec((_R, 128), lambda i: (i, 0))]
    + [pl.BlockSpec((_R, D), lambda i: (i, 0))],
    out_specs=pl.BlockSpec((_R, OUT_D), lambda i: (i, 0)),
    out_shape=jax.ShapeDtypeStruct((B, OUT_D), jnp.float32),
)


def kernel(z_src, z_dst, raw_msg, t_enc, embedding_weight):
    idx = raw_msg.astype(jnp.int32)
    tableT = embedding_weight.T
    tailT = lax.slice(tableT, (0, TAILB), (D, N_NODES))
    pos128 = _sc_stream_gather(idx, tableT, tailT)
    return _concat(z_src, z_dst, pos128, t_enc)
